# baseline (speedup 1.0000x reference)
import functools

import jax
import jax.numpy as jnp
from jax import lax
from jax.experimental import pallas as pl
from jax.experimental.pallas import tpu as pltpu
from jax.experimental.pallas import tpu_sc as plsc

N_TOK = 2048
D = 1024
N_EXP = 8
TOPK = 2

NW = 32
BM = 256
N_HALVES = 2

_MESH = plsc.VectorSubcoreMesh(
    core_axis_name="c", subcore_axis_name="s", num_cores=2, num_subcores=16
)

_ONES16 = lambda: jnp.ones((16,), jnp.int32)
_IOTA16 = lambda: lax.iota(jnp.int32, 16)


def _make_routing(nt):
    npair = nt * TOPK
    ppt = npair // NW
    tpt = nt // NW
    s_pad = npair // BM + N_EXP
    n_align = s_pad * BM

    @functools.partial(
        pl.kernel,
        compiler_params=pltpu.CompilerParams(needs_layout_passes=False),
        out_type=(
            jax.ShapeDtypeStruct((n_align, D), jnp.float32),
            jax.ShapeDtypeStruct((nt,), jnp.int32),
            jax.ShapeDtypeStruct((nt,), jnp.int32),
            jax.ShapeDtypeStruct((16,), jnp.int32),
            jax.ShapeDtypeStruct((s_pad,), jnp.int32),
            jax.ShapeDtypeStruct((s_pad,), jnp.int32),
        ),
        mesh=_MESH,
        scratch_types=(
            pltpu.VMEM((npair,), jnp.int32),
            pltpu.VMEM((16,), jnp.int32),
            pltpu.VMEM((16,), jnp.int32),
            pltpu.VMEM((16,), jnp.int32),
            pltpu.VMEM((tpt,), jnp.int32),
            pltpu.VMEM((tpt,), jnp.int32),
            pltpu.VMEM((16,), jnp.int32),
            pltpu.VMEM((s_pad,), jnp.int32),
            pltpu.VMEM((s_pad,), jnp.int32),
            pltpu.VMEM((tpt, D), jnp.float32),
            pltpu.SemaphoreType.DMA,
            pltpu.SemaphoreType.DMA,
        ),
    )
    def _routing(idx_hbm, x_hbm, xs_hbm, inv0_hbm, inv1_hbm, off_hbm,
                 me_hbm, ee_hbm,
                 keys_v, hist_v, pre_v, cnt_v, inv0_v, inv1_v, off_v,
                 me_v, ee_v, xrows_v, sem0, sem1):
        wid = lax.axis_index("s") * 2 + lax.axis_index("c")
        zeros16 = jnp.zeros((16,), jnp.int32)

        pltpu.sync_copy(idx_hbm, keys_v)
        xcopy = pltpu.async_copy(x_hbm.at[pl.ds(wid * tpt, tpt)], xrows_v, sem0)

        hist_v[...] = zeros16
        pre_v[...] = zeros16

        def _hist_step(i, _):
            k = keys_v[pl.ds(i * 16, 16)]
            plsc.addupdate_scatter(hist_v, [k], _ONES16())
            return 0

        lax.fori_loop(0, npair // 16, _hist_step, 0)

        def _pre_step(i, _):
            k = keys_v[pl.ds(i * 16, 16)]
            plsc.addupdate_scatter(pre_v, [k], _ONES16())
            return 0

        lax.fori_loop(0, wid * (ppt // 16), _pre_step, 0)

        hist = hist_v[...]
        hist_al = ((hist + (BM - 1)) // BM) * BM
        off = jnp.cumsum(hist_al) - hist_al
        cnt_v[...] = off + pre_v[...]

        for v in range(ppt // 16):
            k = keys_v[pl.ds(wid * ppt + v * 16, 16)]
            base = plsc.load_gather(cnt_v, [k])
            rank = zeros16
            for e in range(N_EXP):
                m = k == e
                cs = jnp.cumsum(m.astype(jnp.int32))
                rank = jnp.where(m, cs - 1, rank)
            pos = base + rank
            plsc.addupdate_scatter(cnt_v, [k], _ONES16())
            tok = (_IOTA16() + v * 16) // 2
            even = (_IOTA16() % 2) == 0
            plsc.store_scatter(inv0_v, [tok], pos, mask=even)
            plsc.store_scatter(inv1_v, [tok], pos, mask=jnp.logical_not(even))

        pltpu.sync_copy(inv0_v, inv0_hbm.at[pl.ds(wid * tpt, tpt)])
        pltpu.sync_copy(inv1_v, inv1_hbm.at[pl.ds(wid * tpt, tpt)])

        @pl.when(wid == 0)
        def _():
            off_v[...] = off
            pltpu.sync_copy(off_v, off_hbm)
            lane0 = _IOTA16() == 0
            for sb in range(0, s_pad, 16):
                me_v[pl.ds(min(sb, s_pad - 16), 16)] = zeros16
                ee_v[pl.ds(min(sb, s_pad - 16), 16)] = jnp.full(
                    (16,), 14, jnp.int32)
            s = jnp.int32(0)
            for e in range(N_EXP):
                lane = _IOTA16()
                off_e = jnp.sum(jnp.where(lane == e, off, 0))
                n_e = jnp.sum(jnp.where(lane == e, hist, 0))
                lo = off_e // BM
                hi = lo + (n_e + (BM - 1)) // BM

                def _emit(m, s_, _e=e):
                    plsc.store_scatter(
                        me_v, [jnp.full((16,), s_, jnp.int32)],
                        jnp.full((16,), m, jnp.int32), mask=lane0)
                    plsc.store_scatter(
                        ee_v, [jnp.full((16,), s_, jnp.int32)],
                        jnp.full((16,), _e, jnp.int32), mask=lane0)
                    return s_ + 1

                s = lax.fori_loop(lo, hi, _emit, s)
            pltpu.sync_copy(me_v, me_hbm)
            pltpu.sync_copy(ee_v, ee_hbm)

        xcopy.wait()
        s0 = pltpu.async_copy(xrows_v, xs_hbm.at[inv0_v], sem1)
        s0.wait()
        s1 = pltpu.async_copy(xrows_v, xs_hbm.at[inv1_v], sem1)
        s1.wait()

    return _routing


def _gmm_body(me_ref, ee_ref, x_ref, w_ref, b_ref, y_ref):
    s = pl.program_id(0)
    row0 = me_ref[s] * BM

    @pl.when(ee_ref[s] < N_EXP)
    def _():
        acc = jnp.dot(x_ref[...], w_ref[0], preferred_element_type=jnp.float32)
        y_ref[pl.ds(row0, BM), :] = acc + b_ref[0]


def _make_gmm(nt):
    s_pad = nt * TOPK // BM + N_EXP
    n_align = s_pad * BM

    def _gmm(me, ee, xs, W, b3):
        grid_spec = pltpu.PrefetchScalarGridSpec(
            num_scalar_prefetch=2,
            grid=(s_pad,),
            in_specs=[
                pl.BlockSpec((BM, D), lambda s, me_ref, ee_ref: (me_ref[s], 0)),
                pl.BlockSpec(
                    (1, D, D),
                    lambda s, me_ref, ee_ref: (
                        jnp.minimum(ee_ref[s], N_EXP - 1), 0, 0),
                ),
                pl.BlockSpec(
                    (1, 1, D),
                    lambda s, me_ref, ee_ref: (
                        jnp.minimum(ee_ref[s], N_EXP - 1), 0, 0),
                ),
            ],
            out_specs=pl.BlockSpec(
                (n_align, D), lambda s, me_ref, ee_ref: (0, 0)),
        )
        return pl.pallas_call(
            _gmm_body,
            grid_spec=grid_spec,
            out_shape=jax.ShapeDtypeStruct((n_align, D), jnp.float32),
        )(me, ee, xs, W, b3)

    return _gmm


QT = 16


def _make_combine(nt):
    tpt = nt // NW
    nq = tpt // QT
    n_align = (nt * TOPK // BM + N_EXP) * BM

    @functools.partial(
        pl.kernel,
        compiler_params=pltpu.CompilerParams(needs_layout_passes=False),
        out_type=jax.ShapeDtypeStruct((nt, D), jnp.float32),
        mesh=_MESH,
        scratch_types=(
            pltpu.VMEM((tpt,), jnp.float32),
            pltpu.VMEM((tpt,), jnp.float32),
            pltpu.VMEM((QT,), jnp.int32),
            pltpu.VMEM((QT,), jnp.int32),
            pltpu.VMEM((QT,), jnp.int32),
            pltpu.VMEM((QT,), jnp.int32),
            pltpu.VMEM((QT, D), jnp.float32),
            pltpu.VMEM((QT, D), jnp.float32),
            pltpu.VMEM((QT, D), jnp.float32),
            pltpu.VMEM((QT, D), jnp.float32),
            pltpu.VMEM((QT, D), jnp.float32),
            pltpu.VMEM((QT, D), jnp.float32),
            pltpu.SemaphoreType.DMA,
            pltpu.SemaphoreType.DMA,
            pltpu.SemaphoreType.DMA,
            pltpu.SemaphoreType.DMA,
            pltpu.SemaphoreType.DMA,
            pltpu.SemaphoreType.DMA,
        ),
    )
    def _combine(inv0_hbm, inv1_hbm, p0_hbm, p1_hbm, y_hbm, out_hbm,
                 p0_v, p1_v, iA0, iA1, iB0, iB1, rA0, rA1, rB0, rB1, ocA, ocB,
                 sA0, sA1, sB0, sB1, sOA, sOB):
        wid = lax.axis_index("s") * 2 + lax.axis_index("c")
        base = wid * tpt
        pltpu.sync_copy(p0_hbm.at[pl.ds(base, tpt)], p0_v)
        pltpu.sync_copy(p1_hbm.at[pl.ds(base, tpt)], p1_v)

        bufs = [(iA0, iA1, rA0, rA1, ocA, sA0, sA1, sOA),
                (iB0, iB1, rB0, rB1, ocB, sB0, sB1, sOB)]

        def _issue(q):
            i0, i1, r0, r1, _, s0, s1, _ = bufs[q % 2]
            pltpu.sync_copy(inv0_hbm.at[pl.ds(base + q * QT, QT)], i0)
            pltpu.sync_copy(inv1_hbm.at[pl.ds(base + q * QT, QT)], i1)
            return (pltpu.async_copy(y_hbm.at[i0], r0, s0),
                    pltpu.async_copy(y_hbm.at[i1], r1, s1))

        gath = {0: _issue(0)}
        pend = [None, None]
        for q in range(nq):
            i0, i1, r0, r1, oc, s0, s1, sO = bufs[q % 2]
            if q + 1 < nq:
                gath[q + 1] = _issue(q + 1)
            for g in gath.pop(q):
                g.wait()
            if pend[q % 2] is not None:
                pend[q % 2].wait()

            def _fma(t, _, _q=q):
                sel = jnp.full((16,), _q * QT + t, jnp.int32)
                g0 = plsc.load_gather(p0_v, [sel])
                g1 = plsc.load_gather(p1_v, [sel])
                for c in range(D // 16):
                    oc[t, pl.ds(c * 16, 16)] = (
                        g0 * r0[t, pl.ds(c * 16, 16)]
                        + g1 * r1[t, pl.ds(c * 16, 16)]
                    )
                return 0

            lax.fori_loop(0, QT, _fma, 0)
            pend[q % 2] = pltpu.async_copy(
                oc, out_hbm.at[pl.ds(base + q * QT, QT)], sO)
        for p in pend:
            if p is not None:
                p.wait()

    return _combine


NT_H = N_TOK // N_HALVES
_routing_h = _make_routing(NT_H)
_gmm_h = _make_gmm(NT_H)
_combine_h = _make_combine(NT_H)


def kernel(input_batch, probabilities, indices, W, b):
    idx32 = indices.reshape(N_TOK * TOPK).astype(jnp.int32)
    p0 = probabilities[:, 0]
    p1 = probabilities[:, 1]
    b3 = b.reshape(N_EXP, 1, D)
    np_h = NT_H * TOPK

    routed = [
        _routing_h(
            lax.slice_in_dim(idx32, h * np_h, (h + 1) * np_h),
            lax.slice_in_dim(input_batch, h * NT_H, (h + 1) * NT_H),
        )
        for h in range(N_HALVES)
    ]
    ys = [_gmm_h(r[4], r[5], r[0], W, b3) for r in routed]
    outs = [
        _combine_h(
            routed[h][1], routed[h][2],
            lax.slice_in_dim(p0, h * NT_H, (h + 1) * NT_H),
            lax.slice_in_dim(p1, h * NT_H, (h + 1) * NT_H),
            ys[h],
        )
        for h in range(N_HALVES)
    ]
    out = jnp.concatenate(outs, axis=0)
    total_loss = jnp.zeros((), dtype=jnp.float32)
    return (out, total_loss)


# speedup vs baseline: 1.2594x; 1.2594x over previous
import functools

import jax
import jax.numpy as jnp
from jax import lax
from jax.experimental import pallas as pl
from jax.experimental.pallas import tpu as pltpu
from jax.experimental.pallas import tpu_sc as plsc

N_TOK = 2048
D = 1024
N_EXP = 8
TOPK = 2
N_PAIR = N_TOK * TOPK

NW = 32
PAIRS_PER_TILE = N_PAIR // NW
TOK_PER_TILE = N_TOK // NW
HALF = TOK_PER_TILE // 2

BM = 256
S_PAD = 24
N_ALIGN = S_PAD * BM

_MESH = plsc.VectorSubcoreMesh(
    core_axis_name="c", subcore_axis_name="s", num_cores=2, num_subcores=16
)

_ONES16 = lambda: jnp.ones((16,), jnp.int32)
_IOTA16 = lambda: lax.iota(jnp.int32, 16)


@functools.partial(
    pl.kernel,
    compiler_params=pltpu.CompilerParams(needs_layout_passes=False),
    out_type=(
        jax.ShapeDtypeStruct((N_ALIGN, D), jnp.float32),
        jax.ShapeDtypeStruct((N_TOK,), jnp.int32),
        jax.ShapeDtypeStruct((N_TOK,), jnp.int32),
        jax.ShapeDtypeStruct((16,), jnp.int32),
        jax.ShapeDtypeStruct((S_PAD,), jnp.int32),
        jax.ShapeDtypeStruct((S_PAD,), jnp.int32),
    ),
    mesh=_MESH,
    scratch_types=(
        pltpu.VMEM((N_PAIR,), jnp.int32),
        pltpu.VMEM((16,), jnp.int32),
        pltpu.VMEM((16,), jnp.int32),
        pltpu.VMEM((16,), jnp.int32),
        pltpu.VMEM((TOK_PER_TILE,), jnp.int32),
        pltpu.VMEM((TOK_PER_TILE,), jnp.int32),
        pltpu.VMEM((16,), jnp.int32),
        pltpu.VMEM((S_PAD,), jnp.int32),
        pltpu.VMEM((S_PAD,), jnp.int32),
        pltpu.VMEM((TOK_PER_TILE, D), jnp.float32),
        pltpu.SemaphoreType.DMA,
        pltpu.SemaphoreType.DMA,
    ),
)
def _routing(idx_hbm, x_hbm, xs_hbm, inv0_hbm, inv1_hbm, off_hbm,
             me_hbm, ee_hbm,
             keys_v, hist_v, pre_v, cnt_v, inv0_v, inv1_v, off_v, me_v, ee_v,
             xrows_v, sem0, sem1):
    wid = lax.axis_index("s") * 2 + lax.axis_index("c")
    zeros16 = jnp.zeros((16,), jnp.int32)

    pltpu.sync_copy(idx_hbm, keys_v)
    xcopy = pltpu.async_copy(x_hbm.at[pl.ds(wid * TOK_PER_TILE, TOK_PER_TILE)],
                             xrows_v, sem0)

    hist_v[...] = zeros16
    pre_v[...] = zeros16

    def _hist_step(i, _):
        k = keys_v[pl.ds(i * 16, 16)]
        plsc.addupdate_scatter(hist_v, [k], _ONES16())
        return 0

    lax.fori_loop(0, N_PAIR // 16, _hist_step, 0)

    def _pre_step(i, _):
        k = keys_v[pl.ds(i * 16, 16)]
        plsc.addupdate_scatter(pre_v, [k], _ONES16())
        return 0

    lax.fori_loop(0, wid * (PAIRS_PER_TILE // 16), _pre_step, 0)

    hist = hist_v[...]
    hist_al = ((hist + (BM - 1)) // BM) * BM
    off = jnp.cumsum(hist_al) - hist_al
    cnt_v[...] = off + pre_v[...]

    for v in range(PAIRS_PER_TILE // 16):
        k = keys_v[pl.ds(wid * PAIRS_PER_TILE + v * 16, 16)]
        base = plsc.load_gather(cnt_v, [k])
        rank = zeros16
        for e in range(N_EXP):
            m = k == e
            cs = jnp.cumsum(m.astype(jnp.int32))
            rank = jnp.where(m, cs - 1, rank)
        pos = base + rank
        plsc.addupdate_scatter(cnt_v, [k], _ONES16())
        tok = (_IOTA16() + v * 16) // 2
        even = (_IOTA16() % 2) == 0
        plsc.store_scatter(inv0_v, [tok], pos, mask=even)
        plsc.store_scatter(inv1_v, [tok], pos, mask=jnp.logical_not(even))

    pltpu.sync_copy(inv0_v, inv0_hbm.at[pl.ds(wid * TOK_PER_TILE, TOK_PER_TILE)])
    pltpu.sync_copy(inv1_v, inv1_hbm.at[pl.ds(wid * TOK_PER_TILE, TOK_PER_TILE)])

    @pl.when(wid == 0)
    def _():
        off_v[...] = off
        pltpu.sync_copy(off_v, off_hbm)
        lane0 = _IOTA16() == 0
        me_v[pl.ds(0, 16)] = jnp.full((16,), S_PAD - 1, jnp.int32)
        me_v[pl.ds(S_PAD - 16, 16)] = jnp.full((16,), S_PAD - 1, jnp.int32)
        ee_v[pl.ds(0, 16)] = jnp.full((16,), 14, jnp.int32)
        ee_v[pl.ds(S_PAD - 16, 16)] = jnp.full((16,), 14, jnp.int32)
        s = jnp.int32(0)
        for e in range(N_EXP):
            lane = _IOTA16()
            off_e = jnp.sum(jnp.where(lane == e, off, 0))
            n_e = jnp.sum(jnp.where(lane == e, hist, 0))
            lo = off_e // BM
            hi = lo + (n_e + (BM - 1)) // BM

            def _emit(m, s_, _e=e):
                plsc.store_scatter(
                    me_v, [jnp.full((16,), s_, jnp.int32)],
                    jnp.full((16,), m, jnp.int32), mask=lane0)
                plsc.store_scatter(
                    ee_v, [jnp.full((16,), s_, jnp.int32)],
                    jnp.full((16,), _e, jnp.int32), mask=lane0)
                return s_ + 1

            s = lax.fori_loop(lo, hi, _emit, s)
        pltpu.sync_copy(me_v, me_hbm)
        pltpu.sync_copy(ee_v, ee_hbm)

    xcopy.wait()
    s0 = pltpu.async_copy(xrows_v, xs_hbm.at[inv0_v], sem1)
    s0.wait()
    s1 = pltpu.async_copy(xrows_v, xs_hbm.at[inv1_v], sem1)
    s1.wait()


def _gmm_body(me_ref, ee_ref, x_ref, w_ref, b_ref, y_ref):
    s = pl.program_id(0)

    @pl.when(ee_ref[s] < N_EXP)
    def _():
        acc = jnp.dot(
            x_ref[...], w_ref[0], preferred_element_type=jnp.float32,
        )
        y_ref[...] = acc + b_ref[0]


def _gmm(me, ee, xs, Wb, b3):
    grid_spec = pltpu.PrefetchScalarGridSpec(
        num_scalar_prefetch=2,
        grid=(S_PAD,),
        in_specs=[
            pl.BlockSpec((BM, D), lambda s, me_ref, ee_ref: (me_ref[s], 0)),
            pl.BlockSpec(
                (1, D, D),
                lambda s, me_ref, ee_ref: (
                    jnp.minimum(ee_ref[s], N_EXP - 1), 0, 0),
            ),
            pl.BlockSpec(
                (1, 1, D),
                lambda s, me_ref, ee_ref: (
                    jnp.minimum(ee_ref[s], N_EXP - 1), 0, 0),
            ),
        ],
        out_specs=pl.BlockSpec((BM, D), lambda s, me_ref, ee_ref: (me_ref[s], 0)),
    )
    return pl.pallas_call(
        _gmm_body,
        grid_spec=grid_spec,
        out_shape=jax.ShapeDtypeStruct((N_ALIGN, D), jnp.float32),
    )(me, ee, xs, Wb, b3)


QT = 16
NQ = TOK_PER_TILE // QT


@functools.partial(
    pl.kernel,
    compiler_params=pltpu.CompilerParams(needs_layout_passes=False),
    out_type=jax.ShapeDtypeStruct((N_TOK, D), jnp.float32),
    mesh=_MESH,
    scratch_types=(
        pltpu.VMEM((TOK_PER_TILE,), jnp.float32),
        pltpu.VMEM((TOK_PER_TILE,), jnp.float32),
        pltpu.VMEM((QT,), jnp.int32),
        pltpu.VMEM((QT,), jnp.int32),
        pltpu.VMEM((QT,), jnp.int32),
        pltpu.VMEM((QT,), jnp.int32),
        pltpu.VMEM((QT, D), jnp.float32),
        pltpu.VMEM((QT, D), jnp.float32),
        pltpu.VMEM((QT, D), jnp.float32),
        pltpu.VMEM((QT, D), jnp.float32),
        pltpu.VMEM((QT, D), jnp.float32),
        pltpu.VMEM((QT, D), jnp.float32),
        pltpu.SemaphoreType.DMA,
        pltpu.SemaphoreType.DMA,
        pltpu.SemaphoreType.DMA,
        pltpu.SemaphoreType.DMA,
        pltpu.SemaphoreType.DMA,
        pltpu.SemaphoreType.DMA,
    ),
)
def _combine(inv0_hbm, inv1_hbm, p0_hbm, p1_hbm, y_hbm, out_hbm,
             p0_v, p1_v, iA0, iA1, iB0, iB1, rA0, rA1, rB0, rB1, ocA, ocB,
             sA0, sA1, sB0, sB1, sOA, sOB):
    wid = lax.axis_index("s") * 2 + lax.axis_index("c")
    base = wid * TOK_PER_TILE
    pltpu.sync_copy(p0_hbm.at[pl.ds(base, TOK_PER_TILE)], p0_v)
    pltpu.sync_copy(p1_hbm.at[pl.ds(base, TOK_PER_TILE)], p1_v)

    bufs = [(iA0, iA1, rA0, rA1, ocA, sA0, sA1, sOA),
            (iB0, iB1, rB0, rB1, ocB, sB0, sB1, sOB)]

    def _issue(q):
        i0, i1, r0, r1, _, s0, s1, _ = bufs[q % 2]
        pltpu.sync_copy(inv0_hbm.at[pl.ds(base + q * QT, QT)], i0)
        pltpu.sync_copy(inv1_hbm.at[pl.ds(base + q * QT, QT)], i1)
        return (pltpu.async_copy(y_hbm.at[i0], r0, s0),
                pltpu.async_copy(y_hbm.at[i1], r1, s1))

    gath = {0: _issue(0)}
    pend = [None, None]
    for q in range(NQ):
        i0, i1, r0, r1, oc, s0, s1, sO = bufs[q % 2]
        if q + 1 < NQ:
            gath[q + 1] = _issue(q + 1)
        for g in gath.pop(q):
            g.wait()
        if pend[q % 2] is not None:
            pend[q % 2].wait()

        def _fma(t, _, _q=q):
            sel = jnp.full((16,), _q * QT + t, jnp.int32)
            g0 = plsc.load_gather(p0_v, [sel])
            g1 = plsc.load_gather(p1_v, [sel])
            for c in range(D // 16):
                oc[t, pl.ds(c * 16, 16)] = (
                    g0 * r0[t, pl.ds(c * 16, 16)]
                    + g1 * r1[t, pl.ds(c * 16, 16)]
                )
            return 0

        lax.fori_loop(0, QT, _fma, 0)
        pend[q % 2] = pltpu.async_copy(
            oc, out_hbm.at[pl.ds(base + q * QT, QT)], sO)
    for p in pend:
        if p is not None:
            p.wait()


def kernel(input_batch, probabilities, indices, W, b):
    idx32 = indices.reshape(N_PAIR).astype(jnp.int32)
    p0 = probabilities[:, 0]
    p1 = probabilities[:, 1]
    xs, inv0, inv1, off, me, ee = _routing(idx32, input_batch)
    y = _gmm(me, ee, xs, W, b.reshape(N_EXP, 1, D))
    out = _combine(inv0, inv1, p0, p1, y)
    total_loss = jnp.zeros((), dtype=jnp.float32)
    return (out, total_loss)
